# speedup vs baseline: 1.0206x; 1.0206x over previous
import jax
import jax.numpy as jnp
from jax.experimental import pallas as pl

DIM = 128
DEPTH = 2


def _tanh_kernel(x_ref, o_ref):
    o_ref[...] = jnp.tanh(x_ref[...])


def _tanh(x):
    n, d = x.shape
    blk = 1000
    return pl.pallas_call(
        _tanh_kernel,
        grid=(pl.cdiv(n, blk),),
        in_specs=[pl.BlockSpec((blk, d), lambda i: (i, 0))],
        out_specs=pl.BlockSpec((blk, d), lambda i: (i, 0)),
        out_shape=jax.ShapeDtypeStruct((n, d), x.dtype),
    )(x)


def _edge_softmax(att1, dst, num_nodes):
    m = jax.ops.segment_max(att1, dst, num_segments=num_nodes)
    m = jnp.where(jnp.isfinite(m), m, 0.0)
    e = jnp.exp(att1 - m[dst])
    denom = jax.ops.segment_sum(e, dst, num_segments=num_nodes)
    return e / denom[dst]


def _rrea_encoder(features, kernels, src, dst, rels_sum, num_nodes):
    features = _tanh(features)
    outputs = [features]
    for l in range(DEPTH):
        neighs = features[src]
        selfs = features[dst]
        neighs = neighs - 2.0 * jnp.sum(neighs * rels_sum, axis=1, keepdims=True) * rels_sum
        comb = jnp.concatenate([selfs, neighs, rels_sum], axis=1)
        att1 = jnp.squeeze(jnp.matmul(comb, kernels[l]), axis=-1)
        att = _edge_softmax(att1, dst, num_nodes)
        new_feature = neighs * att[:, None]
        features = jax.ops.segment_sum(new_feature, dst, num_segments=num_nodes)
        features = _tanh(features)
        outputs.append(features)
    return jnp.concatenate(outputs, axis=1)


def kernel(edge_index, edge_rel, ent_emb, rel_emb, attn_e, attn_r):
    src = edge_index[0]
    dst = edge_index[1]
    num_nodes = ent_emb.shape[0]
    ones = jnp.ones((src.shape[0],), dtype=ent_emb.dtype)
    deg = jax.ops.segment_sum(ones, dst, num_segments=num_nodes)
    deg = jnp.maximum(deg, 1.0)[:, None]
    ent_feature = jax.ops.segment_sum(ent_emb[src], dst, num_segments=num_nodes) / deg
    rel_feature = jax.ops.segment_sum(rel_emb[edge_rel], dst, num_segments=num_nodes) / deg
    r = rel_emb[edge_rel]
    rels_sum = r / jnp.maximum(jnp.linalg.norm(r, axis=1, keepdims=True), 1e-12)
    out_e = _rrea_encoder(ent_feature, attn_e, src, dst, rels_sum, num_nodes)
    out_r = _rrea_encoder(rel_feature, attn_r, src, dst, rels_sum, num_nodes)
    out = jnp.concatenate([out_e, out_r], axis=-1)
    return out


# speedup vs baseline: 2.9497x; 2.8900x over previous
import functools

import jax
import jax.numpy as jnp
from jax import lax
from jax.experimental import pallas as pl
from jax.experimental.pallas import tpu as pltpu
from jax.experimental.pallas import tpu_sc as plsc

N = 10000
NP = 10240
E = 320000
EP = 327680
NREL = 1000
RP = 1024
D = 128
NC = 2
NS = 16
NW = NC * NS
EPT = EP // NW
CH = 128
NCHUNK = EPT // CH
RPS = NP // NS
DUMMY = N + 1

_MESH = plsc.VectorSubcoreMesh(core_axis_name="c", subcore_axis_name="s")
_SC_PARAMS = pltpu.CompilerParams(needs_layout_passes=False)
f32 = jnp.float32
i32 = jnp.int32


def _pass0_body(with_deg, table, gidx, dsti, zrows, zvec,
                acc_out, den_out,
                gidx_v, dst_v, rows_v, ones_v, acc_sh, den_sh):
    cid = lax.axis_index("c")
    sid = lax.axis_index("s")
    wid = sid * NC + cid
    pltpu.sync_copy(zrows.at[pl.ds(sid * RPS, RPS)],
                    acc_sh.at[pl.ds(sid * RPS, RPS)])
    pltpu.sync_copy(zvec.at[pl.ds(sid * RPS, RPS)],
                    den_sh.at[pl.ds(sid * RPS, RPS)])
    if with_deg:
        for k in range(CH // 16):
            ones_v[pl.ds(k * 16, 16)] = jnp.ones((16,), f32)
    plsc.subcore_barrier()

    def body(c, carry):
        base = wid * EPT + c * CH
        pltpu.sync_copy(gidx.at[pl.ds(base, CH)], gidx_v)
        pltpu.sync_copy(dsti.at[pl.ds(base, CH)], dst_v)
        pltpu.sync_copy(table.at[gidx_v], rows_v)
        pltpu.sync_copy(rows_v, acc_sh.at[dst_v], add=True)
        if with_deg:
            pltpu.sync_copy(ones_v, den_sh.at[dst_v], add=True)
        return carry

    lax.fori_loop(0, NCHUNK, body, 0)
    plsc.subcore_barrier()
    pltpu.sync_copy(acc_sh.at[pl.ds(sid * RPS, RPS)],
                    acc_out.at[cid, pl.ds(sid * RPS, RPS)])
    pltpu.sync_copy(den_sh.at[pl.ds(sid * RPS, RPS)],
                    den_out.at[cid, pl.ds(sid * RPS, RPS)])


def _make_pass0(with_deg, nrows):
    return pl.kernel(
        functools.partial(_pass0_body, with_deg),
        out_type=[jax.ShapeDtypeStruct((NC, NP, D), f32),
                  jax.ShapeDtypeStruct((NC, NP), f32)],
        mesh=_MESH,
        compiler_params=_SC_PARAMS,
        scratch_types=[
            pltpu.VMEM((CH,), i32),
            pltpu.VMEM((CH,), i32),
            pltpu.VMEM((CH, D), f32),
            pltpu.VMEM((CH,), f32),
            pltpu.VMEM_SHARED((NP, D), f32),
            pltpu.VMEM_SHARED((NP,), f32),
        ],
    )


def _layer_body(fmat, gflat, a_s, a_n, btab, ctab, rn,
                srci, dsti, reli, zrows, zvec,
                acc_out, den_out,
                src_v, dst_v, rel_v, gidx_v, d_v, exp_v, q_v, asd_v, ans_v,
                frows_v, rrows_v, b_v, c_v, acc_sh, den_sh):
    cid = lax.axis_index("c")
    sid = lax.axis_index("s")
    wid = sid * NC + cid
    pltpu.sync_copy(btab, b_v)
    pltpu.sync_copy(ctab, c_v)
    pltpu.sync_copy(zrows.at[pl.ds(sid * RPS, RPS)],
                    acc_sh.at[pl.ds(sid * RPS, RPS)])
    pltpu.sync_copy(zvec.at[pl.ds(sid * RPS, RPS)],
                    den_sh.at[pl.ds(sid * RPS, RPS)])
    plsc.subcore_barrier()

    iota16 = lax.iota(i32, 16)

    def body(c, carry):
        base = wid * EPT + c * CH
        pltpu.sync_copy(srci.at[pl.ds(base, CH)], src_v)
        pltpu.sync_copy(dsti.at[pl.ds(base, CH)], dst_v)
        pltpu.sync_copy(reli.at[pl.ds(base, CH)], rel_v)
        for k in range(CH // 16):
            sl = pl.ds(k * 16, 16)
            gidx_v[sl] = src_v[sl] * RP + rel_v[sl]
        pltpu.sync_copy(gflat.at[gidx_v], d_v)
        pltpu.sync_copy(a_s.at[dst_v], asd_v)
        pltpu.sync_copy(a_n.at[src_v], ans_v)
        pltpu.sync_copy(fmat.at[src_v], frows_v)
        pltpu.sync_copy(rn.at[rel_v], rrows_v)
        for k in range(CH // 16):
            sl = pl.ds(k * 16, 16)
            d16 = d_v[sl]
            b16 = plsc.load_gather(b_v, [rel_v[sl]])
            c16 = plsc.load_gather(c_v, [rel_v[sl]])
            e16 = jnp.exp(asd_v[sl] + ans_v[sl] + c16 - 2.0 * d16 * b16)
            exp_v[sl] = e16
            q_v[sl] = 2.0 * e16 * d16
        pltpu.sync_copy(exp_v, den_sh.at[dst_v], add=True)

        def row_body(i, rcarry):
            ridx = lax.broadcast(i, (16,))
            p16 = plsc.load_gather(exp_v, [ridx])
            q16 = plsc.load_gather(q_v, [ridx])
            for j in range(D // 16):
                cidx = iota16 + (j * 16)
                fv = plsc.load_gather(frows_v, [ridx, cidx])
                rv = plsc.load_gather(rrows_v, [ridx, cidx])
                plsc.store_scatter(frows_v, [ridx, cidx],
                                   p16 * fv - q16 * rv)
            return rcarry

        lax.fori_loop(0, CH, row_body, 0)
        pltpu.sync_copy(frows_v, acc_sh.at[dst_v], add=True)
        return carry

    lax.fori_loop(0, NCHUNK, body, 0)
    plsc.subcore_barrier()
    pltpu.sync_copy(acc_sh.at[pl.ds(sid * RPS, RPS)],
                    acc_out.at[cid, pl.ds(sid * RPS, RPS)])
    pltpu.sync_copy(den_sh.at[pl.ds(sid * RPS, RPS)],
                    den_out.at[cid, pl.ds(sid * RPS, RPS)])


_layer_pass = pl.kernel(
    _layer_body,
    out_type=[jax.ShapeDtypeStruct((NC, NP, D), f32),
              jax.ShapeDtypeStruct((NC, NP), f32)],
    mesh=_MESH,
    compiler_params=_SC_PARAMS,
    scratch_types=[
        pltpu.VMEM((CH,), i32),
        pltpu.VMEM((CH,), i32),
        pltpu.VMEM((CH,), i32),
        pltpu.VMEM((CH,), i32),
        pltpu.VMEM((CH,), f32),
        pltpu.VMEM((CH,), f32),
        pltpu.VMEM((CH,), f32),
        pltpu.VMEM((CH,), f32),
        pltpu.VMEM((CH,), f32),
        pltpu.VMEM((CH, D), f32),
        pltpu.VMEM((CH, D), f32),
        pltpu.VMEM((RP,), f32),
        pltpu.VMEM((RP,), f32),
        pltpu.VMEM_SHARED((NP, D), f32),
        pltpu.VMEM_SHARED((NP,), f32),
    ],
)


def _prep_rel_kernel(r_ref, k1_ref, k2_ref, rn_ref, b_ref, c_ref):
    r = r_ref[...]
    nrm = jnp.sqrt(jnp.sum(r * r, axis=1, keepdims=True))
    rn = r / jnp.maximum(nrm, 1e-12)
    rn_ref[...] = rn
    dn = (((1,), (1,)), ((), ()))
    b_ref[...] = lax.dot_general(k1_ref[...], rn, dn,
                                 preferred_element_type=f32)
    c_ref[...] = lax.dot_general(k2_ref[...], rn, dn,
                                 preferred_element_type=f32)


def _prep_rel(rel_pad, k1s, k2s):
    return pl.pallas_call(
        _prep_rel_kernel,
        out_shape=[jax.ShapeDtypeStruct((RP, D), f32),
                   jax.ShapeDtypeStruct((4, RP), f32),
                   jax.ShapeDtypeStruct((4, RP), f32)],
    )(rel_pad, k1s, k2s)


_BLK = 512


def _prep_layer_kernel(use_deg, with_g,
                       acc_ref, den_ref, rn_ref, k01_ref,
                       f_ref, g_ref, a_ref):
    a = acc_ref[0] + acc_ref[1]
    dn = den_ref[0] + den_ref[1]
    if use_deg:
        invd = 1.0 / jnp.maximum(dn, 1.0)
    else:
        invd = jnp.where(dn > 0.0, 1.0 / dn, 0.0)
    fmat = jnp.tanh(a * invd[:, None])
    f_ref[...] = fmat
    if with_g:
        dnums = (((1,), (1,)), ((), ()))
        g_ref[...] = lax.dot_general(fmat, rn_ref[...], dnums,
                                     preferred_element_type=f32)
        a_ref[...] = lax.dot_general(fmat, k01_ref[...], dnums,
                                     preferred_element_type=f32)


def _prep_layer(acc, den, rn, k01, use_deg, with_g):
    grid = NP // _BLK
    outs = [jax.ShapeDtypeStruct((NP, D), f32),
            jax.ShapeDtypeStruct((NP, RP), f32),
            jax.ShapeDtypeStruct((NP, 2), f32)]
    out_specs = [pl.BlockSpec((_BLK, D), lambda i: (i, 0)),
                 pl.BlockSpec((_BLK, RP), lambda i: (i, 0)),
                 pl.BlockSpec((_BLK, 2), lambda i: (i, 0))]
    return pl.pallas_call(
        functools.partial(_prep_layer_kernel, use_deg, with_g),
        grid=(grid,),
        in_specs=[pl.BlockSpec((NC, _BLK, D), lambda i: (0, i, 0)),
                  pl.BlockSpec((NC, _BLK), lambda i: (0, i)),
                  pl.BlockSpec((RP, D), lambda i: (0, 0)),
                  pl.BlockSpec((2, D), lambda i: (0, 0))],
        out_specs=out_specs,
        out_shape=outs,
    )(acc, den, rn, k01)


def kernel(edge_index, edge_rel, ent_emb, rel_emb, attn_e, attn_r):
    src = edge_index[0]
    dst = edge_index[1]
    pad = EP - E
    srcp = jnp.concatenate([src, jnp.zeros((pad,), i32)])
    dstp = jnp.concatenate([dst, jnp.full((pad,), DUMMY, i32)])
    relp = jnp.concatenate([edge_rel, jnp.zeros((pad,), i32)])
    rel_pad = jnp.pad(rel_emb, ((0, RP - NREL), (0, 0)))

    attn = [attn_e, attn_r]
    k1s = jnp.stack([attn[enc][l, 128:256, 0]
                     for enc in range(2) for l in range(2)])
    k2s = jnp.stack([attn[enc][l, 256:384, 0]
                     for enc in range(2) for l in range(2)])
    rn, btab, ctab = _prep_rel(rel_pad, k1s, k2s)

    zrows = jnp.zeros((NP, D), f32)
    zvec = jnp.zeros((NP,), f32)

    pass0_deg = _make_pass0(True, N)
    pass0_nod = _make_pass0(False, NREL)
    acc_e, deg = pass0_deg(ent_emb, srcp, dstp, zrows, zvec)
    acc_r, _ = pass0_nod(rel_emb, relp, dstp, zrows, zvec)

    outs = []
    for enc in range(2):
        acc = acc_e if enc == 0 else acc_r
        den = deg
        use_deg = True
        for l in range(2):
            k01 = jnp.stack([attn[enc][l, 0:128, 0], attn[enc][l, 128:256, 0]])
            fmat, g, a_sn = _prep_layer(acc, den, rn, k01, use_deg, True)
            outs.append(fmat)
            acc, den = _layer_pass(
                fmat, g.reshape(NP * RP), a_sn[:, 0], a_sn[:, 1],
                btab[enc * 2 + l], ctab[enc * 2 + l], rn,
                srcp, dstp, relp, zrows, zvec)
            use_deg = False
        fmat, _, _ = _prep_layer(acc, den, rn,
                                 jnp.zeros((2, D), f32), False, False)
        outs.append(fmat)

    return jnp.concatenate(outs, axis=1)[:N]


# speedup vs baseline: 4.4278x; 1.5011x over previous
import functools

import jax
import jax.numpy as jnp
from jax import lax
from jax.experimental import pallas as pl
from jax.experimental.pallas import tpu as pltpu
from jax.experimental.pallas import tpu_sc as plsc

N = 10000
NP = 10240
E = 320000
EP = 327680
NREL = 1000
RP = 1024
D = 128
NC = 2
NS = 16
NW = NC * NS
EPT = EP // NW
CH = 128
NCHUNK = EPT // CH
NCHUNK0 = EP // NS // CH
TOTCH = EP // CH
RPS = NP // NS
DUMMY = N + 1

_MESH = plsc.VectorSubcoreMesh(core_axis_name="c", subcore_axis_name="s")
_SC_PARAMS = pltpu.CompilerParams(needs_layout_passes=False)
f32 = jnp.float32
i32 = jnp.int32


def _pass0_body(ctab, epack, zrows, zvec,
                acc_out, den_out,
                pack_v, cidx_v, ones_v, rows_v, acc_sh, den_sh, sem):
    cid = lax.axis_index("c")
    sid = lax.axis_index("s")
    pltpu.sync_copy(zrows.at[pl.ds(sid * RPS, RPS)],
                    acc_sh.at[pl.ds(sid * RPS, RPS)])
    pltpu.sync_copy(zvec.at[pl.ds(sid * RPS, RPS)],
                    den_sh.at[pl.ds(sid * RPS, RPS)])
    for k in range(CH // 16):
        ones_v[pl.ds(k * 16, 16)] = jnp.ones((16,), f32)
    plsc.subcore_barrier()

    is_ent = lax.broadcast(cid == 0, (16,))

    def body(c, carry):
        gc = sid * NCHUNK0 + c
        pltpu.sync_copy(epack.at[gc], pack_v)
        for k in range(CH // 16):
            sl = pl.ds(k * 16, 16)
            cidx_v[sl] = jnp.where(is_ent, pack_v[0, sl], pack_v[2, sl] + N)
        pltpu.sync_copy(ctab.at[cidx_v], rows_v)
        pltpu.sync_copy(rows_v, acc_sh.at[pack_v.at[1]], add=True)
        pltpu.sync_copy(ones_v, den_sh.at[pack_v.at[1]], add=True)
        return carry

    lax.fori_loop(0, NCHUNK0, body, 0)
    plsc.subcore_barrier()
    pltpu.sync_copy(acc_sh.at[pl.ds(sid * RPS, RPS)],
                    acc_out.at[cid, pl.ds(sid * RPS, RPS)])
    pltpu.sync_copy(den_sh.at[pl.ds(sid * RPS, RPS)],
                    den_out.at[cid, pl.ds(sid * RPS, RPS)])


_pass0 = pl.kernel(
    _pass0_body,
    out_type=[jax.ShapeDtypeStruct((NC, NP, D), f32),
              jax.ShapeDtypeStruct((NC, NP), f32)],
    mesh=_MESH,
    compiler_params=_SC_PARAMS,
    scratch_types=[
        pltpu.VMEM((4, CH), i32),
        pltpu.VMEM((CH,), i32),
        pltpu.VMEM((CH,), f32),
        pltpu.VMEM((CH, D), f32),
        pltpu.VMEM_SHARED((NP, D), f32),
        pltpu.VMEM_SHARED((NP,), f32),
        pltpu.SemaphoreType.DMA,
    ],
)


def _layer_body(epack, eflat, qflat, fmat, rn, zrows, zvec,
                acc_out, den_out,
                pack_v, e_v, q_v, frows_v, rrows_v, acc_sh, den_sh, sem):
    cid = lax.axis_index("c")
    sid = lax.axis_index("s")
    wid = sid * NC + cid
    pltpu.sync_copy(zrows.at[pl.ds(sid * RPS, RPS)],
                    acc_sh.at[pl.ds(sid * RPS, RPS)])
    pltpu.sync_copy(zvec.at[pl.ds(sid * RPS, RPS)],
                    den_sh.at[pl.ds(sid * RPS, RPS)])
    plsc.subcore_barrier()

    iota16 = lax.iota(i32, 16)

    def body(c, carry):
        gc = wid * NCHUNK + c
        pltpu.sync_copy(epack.at[gc], pack_v)
        cp1 = pltpu.async_copy(eflat.at[pack_v.at[3]], e_v, sem)
        cp2 = pltpu.async_copy(qflat.at[pack_v.at[3]], q_v, sem)
        cp3 = pltpu.async_copy(fmat.at[pack_v.at[0]], frows_v, sem)
        cp4 = pltpu.async_copy(rn.at[pack_v.at[2]], rrows_v, sem)
        cp1.wait()
        cp2.wait()
        cp3.wait()
        cp4.wait()
        pltpu.sync_copy(e_v, den_sh.at[pack_v.at[1]], add=True)

        def row_body(i, rcarry):
            ridx = lax.broadcast(i, (16,))
            p16 = plsc.load_gather(e_v, [ridx])
            q16 = plsc.load_gather(q_v, [ridx])
            for j in range(D // 16):
                cidx = iota16 + (j * 16)
                fv = plsc.load_gather(frows_v, [ridx, cidx])
                rv = plsc.load_gather(rrows_v, [ridx, cidx])
                plsc.store_scatter(frows_v, [ridx, cidx],
                                   p16 * fv - q16 * rv)
            return rcarry

        lax.fori_loop(0, CH, row_body, 0)
        pltpu.sync_copy(frows_v, acc_sh.at[pack_v.at[1]], add=True)
        return carry

    lax.fori_loop(0, NCHUNK, body, 0)
    plsc.subcore_barrier()
    pltpu.sync_copy(acc_sh.at[pl.ds(sid * RPS, RPS)],
                    acc_out.at[cid, pl.ds(sid * RPS, RPS)])
    pltpu.sync_copy(den_sh.at[pl.ds(sid * RPS, RPS)],
                    den_out.at[cid, pl.ds(sid * RPS, RPS)])


_layer_pass = pl.kernel(
    _layer_body,
    out_type=[jax.ShapeDtypeStruct((NC, NP, D), f32),
              jax.ShapeDtypeStruct((NC, NP), f32)],
    mesh=_MESH,
    compiler_params=_SC_PARAMS,
    scratch_types=[
        pltpu.VMEM((4, CH), i32),
        pltpu.VMEM((CH,), f32),
        pltpu.VMEM((CH,), f32),
        pltpu.VMEM((CH, D), f32),
        pltpu.VMEM((CH, D), f32),
        pltpu.VMEM_SHARED((NP, D), f32),
        pltpu.VMEM_SHARED((NP,), f32),
        pltpu.SemaphoreType.DMA,
    ],
)


def _prep_rel_kernel(r_ref, k1_ref, k2_ref, rn_ref, b_ref, c_ref):
    r = r_ref[...]
    nrm = jnp.sqrt(jnp.sum(r * r, axis=1, keepdims=True))
    rn = r / jnp.maximum(nrm, 1e-12)
    rn_ref[...] = rn
    dn = (((1,), (1,)), ((), ()))
    b_ref[...] = lax.dot_general(k1_ref[...], rn, dn,
                                 preferred_element_type=f32)
    c_ref[...] = lax.dot_general(k2_ref[...], rn, dn,
                                 preferred_element_type=f32)


def _prep_rel(rel_pad, k1s, k2s):
    return pl.pallas_call(
        _prep_rel_kernel,
        out_shape=[jax.ShapeDtypeStruct((RP, D), f32),
                   jax.ShapeDtypeStruct((4, RP), f32),
                   jax.ShapeDtypeStruct((4, RP), f32)],
    )(rel_pad, k1s, k2s)


_BLK = 512
_DN = (((1,), (1,)), ((), ()))


def _prep_layer_kernel(use_deg, slot,
                       acc_ref, den_ref, rn_ref, k1_ref, b_ref, c_ref,
                       f_ref, e_ref, q_ref):
    if use_deg:
        a = acc_ref[slot]
        invd = 1.0 / jnp.maximum(den_ref[0], 1.0)
    else:
        a = acc_ref[0] + acc_ref[1]
        dn = den_ref[0] + den_ref[1]
        invd = jnp.where(dn > 0.0, 1.0 / dn, 0.0)
    fmat = jnp.tanh(a * invd[:, None])
    f_ref[...] = fmat
    g = lax.dot_general(fmat, rn_ref[...], _DN, preferred_element_type=f32)
    an = lax.dot_general(fmat, k1_ref[...], _DN, preferred_element_type=f32)
    e = jnp.exp(an + c_ref[...] - 2.0 * g * b_ref[...])
    e_ref[...] = e
    q_ref[...] = 2.0 * e * g


def _prep_layer(acc, den, rn, k1, b, c, use_deg, slot):
    grid = NP // _BLK
    outs = [jax.ShapeDtypeStruct((NP, D), f32),
            jax.ShapeDtypeStruct((NP, RP), f32),
            jax.ShapeDtypeStruct((NP, RP), f32)]
    out_specs = [pl.BlockSpec((_BLK, D), lambda i: (i, 0)),
                 pl.BlockSpec((_BLK, RP), lambda i: (i, 0)),
                 pl.BlockSpec((_BLK, RP), lambda i: (i, 0))]
    return pl.pallas_call(
        functools.partial(_prep_layer_kernel, use_deg, slot),
        grid=(grid,),
        in_specs=[pl.BlockSpec((NC, _BLK, D), lambda i: (0, i, 0)),
                  pl.BlockSpec((NC, _BLK), lambda i: (0, i)),
                  pl.BlockSpec((RP, D), lambda i: (0, 0)),
                  pl.BlockSpec((1, D), lambda i: (0, 0)),
                  pl.BlockSpec((1, RP), lambda i: (0, 0)),
                  pl.BlockSpec((1, RP), lambda i: (0, 0))],
        out_specs=out_specs,
        out_shape=outs,
    )(acc, den, rn, k1, b, c)


def _final_kernel(acc_ref, den_ref, f_ref):
    a = acc_ref[0] + acc_ref[1]
    dn = den_ref[0] + den_ref[1]
    invd = jnp.where(dn > 0.0, 1.0 / dn, 0.0)
    f_ref[...] = jnp.tanh(a * invd[:, None])


def _finalize(acc, den):
    return pl.pallas_call(
        _final_kernel,
        grid=(NP // _BLK,),
        in_specs=[pl.BlockSpec((NC, _BLK, D), lambda i: (0, i, 0)),
                  pl.BlockSpec((NC, _BLK), lambda i: (0, i))],
        out_specs=pl.BlockSpec((_BLK, D), lambda i: (i, 0)),
        out_shape=jax.ShapeDtypeStruct((NP, D), f32),
    )(acc, den)


def kernel(edge_index, edge_rel, ent_emb, rel_emb, attn_e, attn_r):
    src = edge_index[0]
    dst = edge_index[1]
    pad = EP - E
    srcp = jnp.concatenate([src, jnp.zeros((pad,), i32)])
    dstp = jnp.concatenate([dst, jnp.full((pad,), DUMMY, i32)])
    relp = jnp.concatenate([edge_rel, jnp.zeros((pad,), i32)])
    gidxp = srcp * RP + relp
    epack = (jnp.stack([srcp, dstp, relp, gidxp])
             .reshape(4, TOTCH, CH).transpose(1, 0, 2))
    rel_pad = jnp.pad(rel_emb, ((0, RP - NREL), (0, 0)))
    ctab0 = jnp.concatenate([ent_emb, rel_emb])

    attn = [attn_e, attn_r]
    k1s = jnp.stack([attn[enc][l, 128:256, 0]
                     for enc in range(2) for l in range(2)])
    k2s = jnp.stack([attn[enc][l, 256:384, 0]
                     for enc in range(2) for l in range(2)])
    rn, btab, ctab = _prep_rel(rel_pad, k1s, k2s)

    zrows = jnp.zeros((NP, D), f32)
    zvec = jnp.zeros((NP,), f32)

    acc0, den0 = _pass0(ctab0, epack, zrows, zvec)

    outs = []
    for enc in range(2):
        acc, den = acc0, den0
        use_deg = True
        for l in range(2):
            ki = enc * 2 + l
            fmat, emat, qmat = _prep_layer(
                acc, den, rn, k1s[ki][None], btab[ki][None], ctab[ki][None],
                use_deg, enc)
            outs.append(fmat)
            acc, den = _layer_pass(
                epack, emat.reshape(NP * RP), qmat.reshape(NP * RP),
                fmat, rn, zrows, zvec)
            use_deg = False
        outs.append(_finalize(acc, den))

    return jnp.concatenate(outs, axis=1)[:N]


# speedup vs baseline: 5.5262x; 1.2481x over previous
import functools

import jax
import jax.numpy as jnp
from jax import lax
from jax.experimental import pallas as pl
from jax.experimental.pallas import tpu as pltpu
from jax.experimental.pallas import tpu_sc as plsc

N = 10000
NP = 10240
E = 320000
EP = 327680
NREL = 1000
RP = 1024
D = 128
NC = 2
NS = 16
NW = NC * NS
EPT = EP // NW
CH0 = 128
NCHUNK0 = EP // NS // CH0
CHL = 64
NCHUNKL = EPT // CHL
RPS = NP // NS
DUMMY = N + 1

_MESH = plsc.VectorSubcoreMesh(core_axis_name="c", subcore_axis_name="s")
_SC_PARAMS = pltpu.CompilerParams(needs_layout_passes=False)
f32 = jnp.float32
i32 = jnp.int32


def _pass0_body(ctab, epack, zrows, zvec,
                acc_out, den_out,
                pack0, pack1, cidx0, cidx1, ones_v, rows0, rows1,
                acc_sh, den_sh, sg0, sg1, ss0, ss1):
    cid = lax.axis_index("c")
    sid = lax.axis_index("s")
    pltpu.sync_copy(zrows.at[pl.ds(sid * RPS, RPS)],
                    acc_sh.at[pl.ds(sid * RPS, RPS)])
    pltpu.sync_copy(zvec.at[pl.ds(sid * RPS, RPS)],
                    den_sh.at[pl.ds(sid * RPS, RPS)])
    for k in range(CH0 // 16):
        ones_v[pl.ds(k * 16, 16)] = jnp.ones((16,), f32)
    plsc.subcore_barrier()

    is_ent = lax.broadcast(cid == 0, (16,))
    base = sid * NCHUNK0
    bufs = ((pack0, cidx0, rows0, sg0, ss0),
            (pack1, cidx1, rows1, sg1, ss1))

    def fire_gathers(b, c):
        pack, cv, rows, sg, _ = bufs[b]
        pltpu.sync_copy(epack.at[c], pack)
        for k in range(CH0 // 16):
            sl = pl.ds(k * 16, 16)
            cv[sl] = jnp.where(is_ent, pack[0, sl], pack[2, sl] + N)
        pltpu.async_copy(ctab.at[cv], rows, sg)

    def wait_gathers(b):
        pack, cv, rows, sg, _ = bufs[b]
        pltpu.make_async_copy(ctab.at[cv], rows, sg).wait()

    def fire_scatters(b):
        pack, cv, rows, _, ss = bufs[b]
        d1 = pltpu.async_copy(rows, acc_sh.at[pack.at[1]], ss, add=True)
        d2 = pltpu.async_copy(ones_v, den_sh.at[pack.at[1]], ss, add=True)
        return d1, d2

    fire_gathers(0, base)
    fire_gathers(1, base + 1)

    def pair(i, carry):
        c0 = 2 * i
        wait_gathers(0)
        s0 = fire_scatters(0)
        wait_gathers(1)
        s1 = fire_scatters(1)
        s0[0].wait()
        s0[1].wait()
        fire_gathers(0, base + jnp.minimum(c0 + 2, NCHUNK0 - 1))
        s1[0].wait()
        s1[1].wait()
        fire_gathers(1, base + jnp.minimum(c0 + 3, NCHUNK0 - 1))
        return carry

    lax.fori_loop(0, NCHUNK0 // 2, pair, 0)
    wait_gathers(0)
    wait_gathers(1)
    plsc.subcore_barrier()
    pltpu.sync_copy(acc_sh.at[pl.ds(sid * RPS, RPS)],
                    acc_out.at[cid, pl.ds(sid * RPS, RPS)])
    pltpu.sync_copy(den_sh.at[pl.ds(sid * RPS, RPS)],
                    den_out.at[cid, pl.ds(sid * RPS, RPS)])


_pass0 = pl.kernel(
    _pass0_body,
    out_type=[jax.ShapeDtypeStruct((NC, NP, D), f32),
              jax.ShapeDtypeStruct((NC, NP), f32)],
    mesh=_MESH,
    compiler_params=_SC_PARAMS,
    scratch_types=[
        pltpu.VMEM((4, CH0), i32),
        pltpu.VMEM((4, CH0), i32),
        pltpu.VMEM((CH0,), i32),
        pltpu.VMEM((CH0,), i32),
        pltpu.VMEM((CH0,), f32),
        pltpu.VMEM((CH0, D), f32),
        pltpu.VMEM((CH0, D), f32),
        pltpu.VMEM_SHARED((NP, D), f32),
        pltpu.VMEM_SHARED((NP,), f32),
        pltpu.SemaphoreType.DMA,
        pltpu.SemaphoreType.DMA,
        pltpu.SemaphoreType.DMA,
        pltpu.SemaphoreType.DMA,
    ],
)


def _layer_body(epack, eflat, qflat, fmat, rn, zrows, zvec,
                acc_out, den_out,
                pack0, pack1, e0, e1, q0, q1, f0, f1, r0, r1,
                acc_sh, den_sh, sg0, sg1, ss0, ss1):
    cid = lax.axis_index("c")
    sid = lax.axis_index("s")
    wid = sid * NC + cid
    pltpu.sync_copy(zrows.at[pl.ds(sid * RPS, RPS)],
                    acc_sh.at[pl.ds(sid * RPS, RPS)])
    pltpu.sync_copy(zvec.at[pl.ds(sid * RPS, RPS)],
                    den_sh.at[pl.ds(sid * RPS, RPS)])
    plsc.subcore_barrier()

    iota16 = lax.iota(i32, 16)
    base = wid * NCHUNKL
    bufs = ((pack0, e0, q0, f0, r0, sg0, ss0),
            (pack1, e1, q1, f1, r1, sg1, ss1))

    def fire_gathers(b, c):
        pack, ev, qv, fv, rv, sg, _ = bufs[b]
        pltpu.sync_copy(epack.at[c], pack)
        pltpu.async_copy(eflat.at[pack.at[3]], ev, sg)
        pltpu.async_copy(qflat.at[pack.at[3]], qv, sg)
        pltpu.async_copy(fmat.at[pack.at[0]], fv, sg)
        pltpu.async_copy(rn.at[pack.at[2]], rv, sg)

    def wait_gathers(b):
        pack, ev, qv, fv, rv, sg, _ = bufs[b]
        pltpu.make_async_copy(eflat.at[pack.at[3]], ev, sg).wait()
        pltpu.make_async_copy(qflat.at[pack.at[3]], qv, sg).wait()
        pltpu.make_async_copy(fmat.at[pack.at[0]], fv, sg).wait()
        pltpu.make_async_copy(rn.at[pack.at[2]], rv, sg).wait()

    def rowloop(b):
        pack, ev, qv, fv, rv, sg, _ = bufs[b]

        def row_body(i, rc):
            ridx = lax.broadcast(i, (16,))
            p16 = plsc.load_gather(ev, [ridx])
            q16 = plsc.load_gather(qv, [ridx])
            for j in range(D // 16):
                cidx = iota16 + (j * 16)
                fvv = plsc.load_gather(fv, [ridx, cidx])
                rvv = plsc.load_gather(rv, [ridx, cidx])
                plsc.store_scatter(fv, [ridx, cidx],
                                   p16 * fvv - q16 * rvv)
            return rc

        lax.fori_loop(0, CHL, row_body, 0)

    def fire_scatters(b):
        pack, ev, qv, fv, rv, _, ss = bufs[b]
        d1 = pltpu.async_copy(ev, den_sh.at[pack.at[1]], ss, add=True)
        d2 = pltpu.async_copy(fv, acc_sh.at[pack.at[1]], ss, add=True)
        return d1, d2

    fire_gathers(0, base)
    fire_gathers(1, base + 1)

    def pair(i, carry):
        c0 = 2 * i
        wait_gathers(0)
        rowloop(0)
        s0 = fire_scatters(0)
        wait_gathers(1)
        rowloop(1)
        s1 = fire_scatters(1)
        s0[0].wait()
        s0[1].wait()
        fire_gathers(0, base + jnp.minimum(c0 + 2, NCHUNKL - 1))
        s1[0].wait()
        s1[1].wait()
        fire_gathers(1, base + jnp.minimum(c0 + 3, NCHUNKL - 1))
        return carry

    lax.fori_loop(0, NCHUNKL // 2, pair, 0)
    wait_gathers(0)
    wait_gathers(1)
    plsc.subcore_barrier()
    pltpu.sync_copy(acc_sh.at[pl.ds(sid * RPS, RPS)],
                    acc_out.at[cid, pl.ds(sid * RPS, RPS)])
    pltpu.sync_copy(den_sh.at[pl.ds(sid * RPS, RPS)],
                    den_out.at[cid, pl.ds(sid * RPS, RPS)])


_layer_pass = pl.kernel(
    _layer_body,
    out_type=[jax.ShapeDtypeStruct((NC, NP, D), f32),
              jax.ShapeDtypeStruct((NC, NP), f32)],
    mesh=_MESH,
    compiler_params=_SC_PARAMS,
    scratch_types=[
        pltpu.VMEM((4, CHL), i32),
        pltpu.VMEM((4, CHL), i32),
        pltpu.VMEM((CHL,), f32),
        pltpu.VMEM((CHL,), f32),
        pltpu.VMEM((CHL,), f32),
        pltpu.VMEM((CHL,), f32),
        pltpu.VMEM((CHL, D), f32),
        pltpu.VMEM((CHL, D), f32),
        pltpu.VMEM((CHL, D), f32),
        pltpu.VMEM((CHL, D), f32),
        pltpu.VMEM_SHARED((NP, D), f32),
        pltpu.VMEM_SHARED((NP,), f32),
        pltpu.SemaphoreType.DMA,
        pltpu.SemaphoreType.DMA,
        pltpu.SemaphoreType.DMA,
        pltpu.SemaphoreType.DMA,
    ],
)


def _prep_rel_kernel(r_ref, k1_ref, k2_ref, rn_ref, b_ref, c_ref):
    r = r_ref[...]
    nrm = jnp.sqrt(jnp.sum(r * r, axis=1, keepdims=True))
    rn = r / jnp.maximum(nrm, 1e-12)
    rn_ref[...] = rn
    dn = (((1,), (1,)), ((), ()))
    b_ref[...] = lax.dot_general(k1_ref[...], rn, dn,
                                 preferred_element_type=f32)
    c_ref[...] = lax.dot_general(k2_ref[...], rn, dn,
                                 preferred_element_type=f32)


def _prep_rel(rel_pad, k1s, k2s):
    return pl.pallas_call(
        _prep_rel_kernel,
        out_shape=[jax.ShapeDtypeStruct((RP, D), f32),
                   jax.ShapeDtypeStruct((4, RP), f32),
                   jax.ShapeDtypeStruct((4, RP), f32)],
    )(rel_pad, k1s, k2s)


_BLK = 512
_DN = (((1,), (1,)), ((), ()))


def _prep_layer_kernel(use_deg, slot,
                       acc_ref, den_ref, rn_ref, k1_ref, b_ref, c_ref,
                       f_ref, e_ref, q_ref):
    if use_deg:
        a = acc_ref[slot]
        invd = 1.0 / jnp.maximum(den_ref[0], 1.0)
    else:
        a = acc_ref[0] + acc_ref[1]
        dn = den_ref[0] + den_ref[1]
        invd = jnp.where(dn > 0.0, 1.0 / dn, 0.0)
    fmat = jnp.tanh(a * invd[:, None])
    f_ref[...] = fmat
    g = lax.dot_general(fmat, rn_ref[...], _DN, preferred_element_type=f32)
    an = lax.dot_general(fmat, k1_ref[...], _DN, preferred_element_type=f32)
    e = jnp.exp(an + c_ref[...] - 2.0 * g * b_ref[...])
    e_ref[...] = e
    q_ref[...] = 2.0 * e * g


def _prep_layer(acc, den, rn, k1, b, c, use_deg, slot):
    grid = NP // _BLK
    outs = [jax.ShapeDtypeStruct((NP, D), f32),
            jax.ShapeDtypeStruct((NP, RP), f32),
            jax.ShapeDtypeStruct((NP, RP), f32)]
    out_specs = [pl.BlockSpec((_BLK, D), lambda i: (i, 0)),
                 pl.BlockSpec((_BLK, RP), lambda i: (i, 0)),
                 pl.BlockSpec((_BLK, RP), lambda i: (i, 0))]
    return pl.pallas_call(
        functools.partial(_prep_layer_kernel, use_deg, slot),
        grid=(grid,),
        in_specs=[pl.BlockSpec((NC, _BLK, D), lambda i: (0, i, 0)),
                  pl.BlockSpec((NC, _BLK), lambda i: (0, i)),
                  pl.BlockSpec((RP, D), lambda i: (0, 0)),
                  pl.BlockSpec((1, D), lambda i: (0, 0)),
                  pl.BlockSpec((1, RP), lambda i: (0, 0)),
                  pl.BlockSpec((1, RP), lambda i: (0, 0))],
        out_specs=out_specs,
        out_shape=outs,
    )(acc, den, rn, k1, b, c)


def _final_kernel(acc_ref, den_ref, f_ref):
    a = acc_ref[0] + acc_ref[1]
    dn = den_ref[0] + den_ref[1]
    invd = jnp.where(dn > 0.0, 1.0 / dn, 0.0)
    f_ref[...] = jnp.tanh(a * invd[:, None])


def _finalize(acc, den):
    return pl.pallas_call(
        _final_kernel,
        grid=(NP // _BLK,),
        in_specs=[pl.BlockSpec((NC, _BLK, D), lambda i: (0, i, 0)),
                  pl.BlockSpec((NC, _BLK), lambda i: (0, i))],
        out_specs=pl.BlockSpec((_BLK, D), lambda i: (i, 0)),
        out_shape=jax.ShapeDtypeStruct((NP, D), f32),
    )(acc, den)


def kernel(edge_index, edge_rel, ent_emb, rel_emb, attn_e, attn_r):
    src = edge_index[0]
    dst = edge_index[1]
    pad = EP - E
    srcp = jnp.concatenate([src, jnp.zeros((pad,), i32)])
    dstp = jnp.concatenate([dst, jnp.full((pad,), DUMMY, i32)])
    relp = jnp.concatenate([edge_rel, jnp.zeros((pad,), i32)])
    gidxp = srcp * RP + relp
    stacked = jnp.stack([srcp, dstp, relp, gidxp])
    epack0 = stacked.reshape(4, EP // CH0, CH0).transpose(1, 0, 2)
    epackl = stacked.reshape(4, EP // CHL, CHL).transpose(1, 0, 2)
    rel_pad = jnp.pad(rel_emb, ((0, RP - NREL), (0, 0)))
    ctab0 = jnp.concatenate([ent_emb, rel_emb])

    attn = [attn_e, attn_r]
    k1s = jnp.stack([attn[enc][l, 128:256, 0]
                     for enc in range(2) for l in range(2)])
    k2s = jnp.stack([attn[enc][l, 256:384, 0]
                     for enc in range(2) for l in range(2)])
    rn, btab, ctab = _prep_rel(rel_pad, k1s, k2s)

    zrows = jnp.zeros((NP, D), f32)
    zvec = jnp.zeros((NP,), f32)

    acc0, den0 = _pass0(ctab0, epack0, zrows, zvec)

    outs = []
    for enc in range(2):
        acc, den = acc0, den0
        use_deg = True
        for l in range(2):
            ki = enc * 2 + l
            fmat, emat, qmat = _prep_layer(
                acc, den, rn, k1s[ki][None], btab[ki][None], ctab[ki][None],
                use_deg, enc)
            outs.append(fmat)
            acc, den = _layer_pass(
                epackl, emat.reshape(NP * RP), qmat.reshape(NP * RP),
                fmat, rn, zrows, zvec)
            use_deg = False
        outs.append(_finalize(acc, den))

    return jnp.concatenate(outs, axis=1)[:N]
